# baseline (speedup 1.0000x reference)
import functools

import jax
import jax.numpy as jnp
from jax import lax
from jax.experimental import pallas as pl
from jax.experimental.pallas import tpu as pltpu
from jax.experimental.pallas import tpu_sc as plsc

NUM_EXPERTS = 8
HIDDEN = 1024
INTER = 2048
TOKENS = 2048
TOPK = 2

BR = 128
NPAD = TOKENS * TOPK + NUM_EXPERTS * BR
NB = NPAD // BR
IT = 512
NIT = INTER // IT

NC, NS = 2, 16
NW = NC * NS
GCH = 32
CCH = 16

_SC_MESH = dict(core_axis_name="c", subcore_axis_name="s",
                num_cores=NC, num_subcores=NS)


def _routing_metadata(top_k_index):
    e_flat = top_k_index.reshape(-1).astype(jnp.int32)
    tok = jnp.repeat(jnp.arange(TOKENS, dtype=jnp.int32), TOPK)
    onehot = (e_flat[:, None] == jnp.arange(NUM_EXPERTS, dtype=jnp.int32)[None, :])
    ccum = jnp.cumsum(onehot.astype(jnp.int32), axis=0)
    gsize = ccum[-1]
    rank = jnp.take_along_axis(ccum, e_flat[:, None], axis=1)[:, 0] - 1
    pg = ((gsize + BR - 1) // BR) * BR
    pad_off = jnp.concatenate(
        [jnp.zeros((1,), jnp.int32), jnp.cumsum(pg)])[:NUM_EXPERTS]
    pos = pad_off[e_flat] + rank
    token_pad = (jnp.arange(NPAD, dtype=jnp.int32) % TOKENS).at[pos].set(tok)
    pad_end = pad_off + pg
    blk_start = jnp.arange(NB, dtype=jnp.int32) * BR
    block_expert = jnp.minimum(
        jnp.sum((blk_start[:, None] >= pad_end[None, :]).astype(jnp.int32), axis=1),
        NUM_EXPERTS - 1)
    return pos, token_pad, block_expert


def _sc_gather(hidden_states, token_pad):
    rows_per_w = NPAD // NW
    nch = rows_per_w // GCH

    @functools.partial(
        pl.kernel,
        out_type=jax.ShapeDtypeStruct((NPAD, HIDDEN), jnp.float32),
        mesh=plsc.VectorSubcoreMesh(**_SC_MESH),
        scratch_types=(
            [pltpu.VMEM((rows_per_w,), jnp.int32)]
            + [pltpu.VMEM((GCH, HIDDEN), jnp.float32) for _ in range(3)]
            + [pltpu.SemaphoreType.DMA for _ in range(6)]
        ),
    )
    def gather_k(table_hbm, idx_hbm, out_hbm, idx_v, b0, b1, b2,
                 g0, g1, g2, o0, o1, o2):
        bufs = (b0, b1, b2)
        gsem = (g0, g1, g2)
        osem = (o0, o1, o2)
        wid = lax.axis_index("s") * NC + lax.axis_index("c")
        base = wid * rows_per_w
        pltpu.sync_copy(idx_hbm.at[pl.ds(base, rows_per_w)], idx_v)
        gcp = [None] * nch
        ocp = [None] * nch
        for ci in range(min(3, nch)):
            gcp[ci] = pltpu.async_copy(
                table_hbm.at[idx_v.at[pl.ds(ci * GCH, GCH)]], bufs[ci % 3],
                gsem[ci % 3])
        for ci in range(nch):
            gcp[ci].wait()
            ocp[ci] = pltpu.async_copy(
                bufs[ci % 3], out_hbm.at[pl.ds(base + ci * GCH, GCH)],
                osem[ci % 3])
            nxt = ci + 3
            if nxt < nch:
                ocp[ci].wait()
                gcp[nxt] = pltpu.async_copy(
                    table_hbm.at[idx_v.at[pl.ds(nxt * GCH, GCH)]],
                    bufs[nxt % 3], gsem[nxt % 3])
        for ci in range(max(0, nch - 3), nch):
            ocp[ci].wait()

    return gather_k(hidden_states, token_pad)


def _sc_gather_pair(y_sorted, pos0, pos1):
    tok_per_w = TOKENS // NW
    nj = 2 * (tok_per_w // GCH)

    @functools.partial(
        pl.kernel,
        out_type=[jax.ShapeDtypeStruct((TOKENS, HIDDEN), jnp.float32),
                  jax.ShapeDtypeStruct((TOKENS, HIDDEN), jnp.float32)],
        mesh=plsc.VectorSubcoreMesh(**_SC_MESH),
        scratch_types=(
            [pltpu.VMEM((tok_per_w,), jnp.int32) for _ in range(2)]
            + [pltpu.VMEM((GCH, HIDDEN), jnp.float32) for _ in range(3)]
            + [pltpu.SemaphoreType.DMA for _ in range(6)]
        ),
    )
    def pair_k(y_hbm, p0_hbm, p1_hbm, out0_hbm, out1_hbm, p0_v, p1_v,
               b0, b1, b2, g0, g1, g2, o0, o1, o2):
        bufs = (b0, b1, b2)
        gsem = (g0, g1, g2)
        osem = (o0, o1, o2)
        wid = lax.axis_index("s") * NC + lax.axis_index("c")
        base = wid * tok_per_w
        pltpu.sync_copy(p0_hbm.at[pl.ds(base, tok_per_w)], p0_v)
        pltpu.sync_copy(p1_hbm.at[pl.ds(base, tok_per_w)], p1_v)
        pv = (p0_v, p1_v)

        def job(j):
            return pv[j % 2], (j // 2) * GCH

        def start_gather(j):
            idx_v, toff = job(j)
            return pltpu.async_copy(
                y_hbm.at[idx_v.at[pl.ds(toff, GCH)]], bufs[j % 3], gsem[j % 3])

        gcp = [None] * nj
        ocp = [None] * nj
        for j in range(min(3, nj)):
            gcp[j] = start_gather(j)
        for j in range(nj):
            idx_v, toff = job(j)
            out_hbm = (out0_hbm, out1_hbm)[j % 2]
            gcp[j].wait()
            ocp[j] = pltpu.async_copy(
                bufs[j % 3], out_hbm.at[pl.ds(base + toff, GCH)], osem[j % 3])
            nxt = j + 3
            if nxt < nj:
                ocp[j].wait()
                gcp[nxt] = start_gather(nxt)
        for j in range(max(0, nj - 3), nj):
            ocp[j].wait()

    return pair_k(y_sorted, pos0, pos1)


def _add_body(a_ref, b_ref, wa_ref, wb_ref, o_ref):
    o_ref[...] = (a_ref[...] * wa_ref[:, 0:1] + b_ref[...] * wb_ref[:, 0:1])


def _tc_add(a, b, wa, wb):
    blk = 256
    return pl.pallas_call(
        _add_body,
        grid=(TOKENS // blk,),
        in_specs=[pl.BlockSpec((blk, HIDDEN), lambda i: (i, 0)),
                  pl.BlockSpec((blk, HIDDEN), lambda i: (i, 0)),
                  pl.BlockSpec((blk, 128), lambda i: (i, 0)),
                  pl.BlockSpec((blk, 128), lambda i: (i, 0))],
        out_specs=pl.BlockSpec((blk, HIDDEN), lambda i: (i, 0)),
        out_shape=jax.ShapeDtypeStruct((TOKENS, HIDDEN), jnp.float32),
    )(a, b, wa, wb)


def _fused_body(be_ref, x_ref, gw_ref, uw_ref, dw_ref, y_ref):
    x = x_ref[...]
    dn = (((1,), (1,)), ((), ()))
    acc = jnp.zeros((BR, HIDDEN), jnp.float32)
    for c in range(NIT):
        gw = gw_ref[0, 0, c * IT:(c + 1) * IT, :]
        uw = uw_ref[0, 0, c * IT:(c + 1) * IT, :]
        g = lax.dot_general(x, gw, dn, preferred_element_type=jnp.float32)
        u = lax.dot_general(x, uw, dn, preferred_element_type=jnp.float32)
        h = (g * lax.logistic(g)) * u
        dwc = dw_ref[0, :, c * IT:(c + 1) * IT]
        acc = acc + lax.dot_general(h, dwc, dn, preferred_element_type=jnp.float32)
    y_ref[...] = acc


def _grouped_mlp(x_sorted, gup4, down_proj, block_expert):
    return pl.pallas_call(
        _fused_body,
        grid_spec=pltpu.PrefetchScalarGridSpec(
            num_scalar_prefetch=1,
            grid=(NB,),
            in_specs=[
                pl.BlockSpec((BR, HIDDEN), lambda b, be: (b, 0)),
                pl.BlockSpec((1, 1, INTER, HIDDEN), lambda b, be: (be[b], 0, 0, 0)),
                pl.BlockSpec((1, 1, INTER, HIDDEN), lambda b, be: (be[b], 1, 0, 0)),
                pl.BlockSpec((1, HIDDEN, INTER), lambda b, be: (be[b], 0, 0)),
            ],
            out_specs=pl.BlockSpec((BR, HIDDEN), lambda b, be: (b, 0)),
        ),
        out_shape=jax.ShapeDtypeStruct((NPAD, HIDDEN), jnp.float32),
    )(block_expert, x_sorted, gup4, gup4, down_proj)


def kernel(hidden_states, top_k_index, top_k_weights, gate_up_proj, down_proj):
    pos, token_pad, block_expert = _routing_metadata(top_k_index)
    w32 = top_k_weights.astype(jnp.float32)
    wa = jnp.broadcast_to(w32[:, 0:1], (TOKENS, 128))
    wb = jnp.broadcast_to(w32[:, 1:2], (TOKENS, 128))
    gup4 = gate_up_proj.reshape(NUM_EXPERTS, 2, INTER, HIDDEN)

    x_sorted = _sc_gather(hidden_states, token_pad)
    y = _grouped_mlp(x_sorted, gup4, down_proj, block_expert)
    pos2 = pos.reshape(TOKENS, TOPK)
    c0, c1 = _sc_gather_pair(y, pos2[:, 0], pos2[:, 1])
    return _tc_add(c0, c1, wa, wb)


# speedup vs baseline: 1.3387x; 1.3387x over previous
import functools

import jax
import jax.numpy as jnp
from jax import lax
from jax.experimental import pallas as pl
from jax.experimental.pallas import tpu as pltpu
from jax.experimental.pallas import tpu_sc as plsc

NUM_EXPERTS = 8
HIDDEN = 1024
INTER = 2048
TOKENS = 2048
TOPK = 2

BR = 256
NPAD = TOKENS * TOPK + NUM_EXPERTS * BR
NB = NPAD // BR
IT = 512
NIT = INTER // IT

NC, NS = 2, 16
NW = NC * NS
GCH = 32
CCH = 16

_SC_MESH = dict(core_axis_name="c", subcore_axis_name="s",
                num_cores=NC, num_subcores=NS)


def _routing_metadata(top_k_index):
    e_flat = top_k_index.reshape(-1).astype(jnp.int32)
    tok = jnp.repeat(jnp.arange(TOKENS, dtype=jnp.int32), TOPK)
    onehot = (e_flat[:, None] == jnp.arange(NUM_EXPERTS, dtype=jnp.int32)[None, :])
    ccum = jnp.cumsum(onehot.astype(jnp.int32), axis=0)
    gsize = ccum[-1]
    rank = jnp.take_along_axis(ccum, e_flat[:, None], axis=1)[:, 0] - 1
    pg = ((gsize + BR - 1) // BR) * BR
    pad_off = jnp.concatenate(
        [jnp.zeros((1,), jnp.int32), jnp.cumsum(pg)])[:NUM_EXPERTS]
    pos = pad_off[e_flat] + rank
    token_pad = (jnp.arange(NPAD, dtype=jnp.int32) % TOKENS).at[pos].set(tok)
    pad_end = pad_off + pg
    blk_start = jnp.arange(NB, dtype=jnp.int32) * BR
    block_expert = jnp.minimum(
        jnp.sum((blk_start[:, None] >= pad_end[None, :]).astype(jnp.int32), axis=1),
        NUM_EXPERTS - 1)
    return pos, token_pad, block_expert


def _sc_gather(hidden_states, token_pad):
    rows_per_w = NPAD // NW
    nch = rows_per_w // GCH

    @functools.partial(
        pl.kernel,
        out_type=jax.ShapeDtypeStruct((NPAD, HIDDEN), jnp.float32),
        mesh=plsc.VectorSubcoreMesh(**_SC_MESH),
        scratch_types=(
            [pltpu.VMEM((rows_per_w,), jnp.int32)]
            + [pltpu.VMEM((GCH, HIDDEN), jnp.float32) for _ in range(3)]
            + [pltpu.SemaphoreType.DMA for _ in range(6)]
        ),
    )
    def gather_k(table_hbm, idx_hbm, out_hbm, idx_v, b0, b1, b2,
                 g0, g1, g2, o0, o1, o2):
        bufs = (b0, b1, b2)
        gsem = (g0, g1, g2)
        osem = (o0, o1, o2)
        wid = lax.axis_index("s") * NC + lax.axis_index("c")
        base = wid * rows_per_w
        pltpu.sync_copy(idx_hbm.at[pl.ds(base, rows_per_w)], idx_v)
        gcp = [None] * nch
        ocp = [None] * nch
        for ci in range(min(3, nch)):
            gcp[ci] = pltpu.async_copy(
                table_hbm.at[idx_v.at[pl.ds(ci * GCH, GCH)]], bufs[ci % 3],
                gsem[ci % 3])
        for ci in range(nch):
            gcp[ci].wait()
            ocp[ci] = pltpu.async_copy(
                bufs[ci % 3], out_hbm.at[pl.ds(base + ci * GCH, GCH)],
                osem[ci % 3])
            nxt = ci + 3
            if nxt < nch:
                ocp[ci].wait()
                gcp[nxt] = pltpu.async_copy(
                    table_hbm.at[idx_v.at[pl.ds(nxt * GCH, GCH)]],
                    bufs[nxt % 3], gsem[nxt % 3])
        for ci in range(max(0, nch - 3), nch):
            ocp[ci].wait()

    return gather_k(hidden_states, token_pad)


def _sc_gather_pair(y_sorted, pos0, pos1):
    tok_per_w = TOKENS // NW
    nj = 2 * (tok_per_w // GCH)

    @functools.partial(
        pl.kernel,
        out_type=[jax.ShapeDtypeStruct((TOKENS, HIDDEN), jnp.float32),
                  jax.ShapeDtypeStruct((TOKENS, HIDDEN), jnp.float32)],
        mesh=plsc.VectorSubcoreMesh(**_SC_MESH),
        scratch_types=(
            [pltpu.VMEM((tok_per_w,), jnp.int32) for _ in range(2)]
            + [pltpu.VMEM((GCH, HIDDEN), jnp.float32) for _ in range(3)]
            + [pltpu.SemaphoreType.DMA for _ in range(6)]
        ),
    )
    def pair_k(y_hbm, p0_hbm, p1_hbm, out0_hbm, out1_hbm, p0_v, p1_v,
               b0, b1, b2, g0, g1, g2, o0, o1, o2):
        bufs = (b0, b1, b2)
        gsem = (g0, g1, g2)
        osem = (o0, o1, o2)
        wid = lax.axis_index("s") * NC + lax.axis_index("c")
        base = wid * tok_per_w
        pltpu.sync_copy(p0_hbm.at[pl.ds(base, tok_per_w)], p0_v)
        pltpu.sync_copy(p1_hbm.at[pl.ds(base, tok_per_w)], p1_v)
        pv = (p0_v, p1_v)

        def job(j):
            return pv[j % 2], (j // 2) * GCH

        def start_gather(j):
            idx_v, toff = job(j)
            return pltpu.async_copy(
                y_hbm.at[idx_v.at[pl.ds(toff, GCH)]], bufs[j % 3], gsem[j % 3])

        gcp = [None] * nj
        ocp = [None] * nj
        for j in range(min(3, nj)):
            gcp[j] = start_gather(j)
        for j in range(nj):
            idx_v, toff = job(j)
            out_hbm = (out0_hbm, out1_hbm)[j % 2]
            gcp[j].wait()
            ocp[j] = pltpu.async_copy(
                bufs[j % 3], out_hbm.at[pl.ds(base + toff, GCH)], osem[j % 3])
            nxt = j + 3
            if nxt < nj:
                ocp[j].wait()
                gcp[nxt] = start_gather(nxt)
        for j in range(max(0, nj - 3), nj):
            ocp[j].wait()

    return pair_k(y_sorted, pos0, pos1)


def _add_body(a_ref, b_ref, wa_ref, wb_ref, o_ref):
    o_ref[...] = (a_ref[...] * wa_ref[:, 0:1] + b_ref[...] * wb_ref[:, 0:1])


def _tc_add(a, b, wa, wb):
    blk = 256
    return pl.pallas_call(
        _add_body,
        grid=(TOKENS // blk,),
        in_specs=[pl.BlockSpec((blk, HIDDEN), lambda i: (i, 0)),
                  pl.BlockSpec((blk, HIDDEN), lambda i: (i, 0)),
                  pl.BlockSpec((blk, 128), lambda i: (i, 0)),
                  pl.BlockSpec((blk, 128), lambda i: (i, 0))],
        out_specs=pl.BlockSpec((blk, HIDDEN), lambda i: (i, 0)),
        out_shape=jax.ShapeDtypeStruct((TOKENS, HIDDEN), jnp.float32),
    )(a, b, wa, wb)


def _fused_body(be_ref, x_ref, gw_ref, uw_ref, dw_ref, y_ref):
    x = x_ref[...].astype(jnp.bfloat16)
    dn = (((1,), (1,)), ((), ()))
    acc = jnp.zeros((BR, HIDDEN), jnp.float32)
    for c in range(NIT):
        gw = gw_ref[0, 0, c * IT:(c + 1) * IT, :].astype(jnp.bfloat16)
        uw = uw_ref[0, 0, c * IT:(c + 1) * IT, :].astype(jnp.bfloat16)
        g = lax.dot_general(x, gw, dn, preferred_element_type=jnp.float32)
        u = lax.dot_general(x, uw, dn, preferred_element_type=jnp.float32)
        h = ((g * lax.logistic(g)) * u).astype(jnp.bfloat16)
        dwc = dw_ref[0, :, c * IT:(c + 1) * IT].astype(jnp.bfloat16)
        acc = acc + lax.dot_general(h, dwc, dn, preferred_element_type=jnp.float32)
    y_ref[...] = acc


def _grouped_mlp(x_sorted, gup4, down_proj, block_expert):
    return pl.pallas_call(
        _fused_body,
        grid_spec=pltpu.PrefetchScalarGridSpec(
            num_scalar_prefetch=1,
            grid=(NB,),
            in_specs=[
                pl.BlockSpec((BR, HIDDEN), lambda b, be: (b, 0)),
                pl.BlockSpec((1, 1, INTER, HIDDEN), lambda b, be: (be[b], 0, 0, 0)),
                pl.BlockSpec((1, 1, INTER, HIDDEN), lambda b, be: (be[b], 1, 0, 0)),
                pl.BlockSpec((1, HIDDEN, INTER), lambda b, be: (be[b], 0, 0)),
            ],
            out_specs=pl.BlockSpec((BR, HIDDEN), lambda b, be: (b, 0)),
        ),
        out_shape=jax.ShapeDtypeStruct((NPAD, HIDDEN), jnp.float32),
    )(block_expert, x_sorted, gup4, gup4, down_proj)


def kernel(hidden_states, top_k_index, top_k_weights, gate_up_proj, down_proj):
    pos, token_pad, block_expert = _routing_metadata(top_k_index)
    w32 = top_k_weights.astype(jnp.float32)
    wa = jnp.broadcast_to(w32[:, 0:1], (TOKENS, 128))
    wb = jnp.broadcast_to(w32[:, 1:2], (TOKENS, 128))
    gup4 = gate_up_proj.reshape(NUM_EXPERTS, 2, INTER, HIDDEN)

    x_sorted = _sc_gather(hidden_states, token_pad)
    y = _grouped_mlp(x_sorted, gup4, down_proj, block_expert)
    pos2 = pos.reshape(TOKENS, TOPK)
    c0, c1 = _sc_gather_pair(y, pos2[:, 0], pos2[:, 1])
    return _tc_add(c0, c1, wa, wb)


# speedup vs baseline: 1.4185x; 1.0596x over previous
import functools

import jax
import jax.numpy as jnp
from jax import lax
from jax.experimental import pallas as pl
from jax.experimental.pallas import tpu as pltpu
from jax.experimental.pallas import tpu_sc as plsc

NUM_EXPERTS = 8
HIDDEN = 1024
INTER = 2048
TOKENS = 2048
TOPK = 2

BR = 256
NPAD = TOKENS * TOPK + NUM_EXPERTS * BR
NB = NPAD // BR
IT = 512
NIT = INTER // IT

NC, NS = 2, 16
NW = NC * NS
GCH = 32
CCH = 16

_SC_MESH = dict(core_axis_name="c", subcore_axis_name="s",
                num_cores=NC, num_subcores=NS)


def _routing_metadata(top_k_index):
    e_flat = top_k_index.reshape(-1).astype(jnp.int32)
    tok = jnp.repeat(jnp.arange(TOKENS, dtype=jnp.int32), TOPK)
    onehot = (e_flat[:, None] == jnp.arange(NUM_EXPERTS, dtype=jnp.int32)[None, :])
    ccum = jnp.cumsum(onehot.astype(jnp.int32), axis=0)
    gsize = ccum[-1]
    rank = jnp.take_along_axis(ccum, e_flat[:, None], axis=1)[:, 0] - 1
    pg = ((gsize + BR - 1) // BR) * BR
    pad_off = jnp.concatenate(
        [jnp.zeros((1,), jnp.int32), jnp.cumsum(pg)])[:NUM_EXPERTS]
    pos = pad_off[e_flat] + rank
    token_pad = (jnp.arange(NPAD, dtype=jnp.int32) % TOKENS).at[pos].set(tok)
    pad_end = pad_off + pg
    blk_start = jnp.arange(NB, dtype=jnp.int32) * BR
    block_expert = jnp.minimum(
        jnp.sum((blk_start[:, None] >= pad_end[None, :]).astype(jnp.int32), axis=1),
        NUM_EXPERTS - 1)
    nact = jnp.maximum(pad_end[NUM_EXPERTS - 1] // BR, 1)
    bidx = jnp.arange(NB, dtype=jnp.int32)
    active = (bidx < nact).astype(jnp.int32)
    xblk = jnp.minimum(bidx, nact - 1)
    bexp = jnp.where(bidx < nact, block_expert,
                     block_expert[jnp.maximum(nact - 1, 0)])
    block_meta = jnp.stack([bexp, xblk, active])
    return pos, token_pad, block_meta


def _sc_gather(hidden_states, token_pad):
    rows_per_w = NPAD // NW
    nch = rows_per_w // GCH

    @functools.partial(
        pl.kernel,
        out_type=jax.ShapeDtypeStruct((NPAD, HIDDEN), jnp.float32),
        mesh=plsc.VectorSubcoreMesh(**_SC_MESH),
        scratch_types=(
            [pltpu.VMEM((rows_per_w,), jnp.int32)]
            + [pltpu.VMEM((GCH, HIDDEN), jnp.float32) for _ in range(3)]
            + [pltpu.SemaphoreType.DMA for _ in range(6)]
        ),
    )
    def gather_k(table_hbm, idx_hbm, out_hbm, idx_v, b0, b1, b2,
                 g0, g1, g2, o0, o1, o2):
        bufs = (b0, b1, b2)
        gsem = (g0, g1, g2)
        osem = (o0, o1, o2)
        wid = lax.axis_index("s") * NC + lax.axis_index("c")
        base = wid * rows_per_w
        pltpu.sync_copy(idx_hbm.at[pl.ds(base, rows_per_w)], idx_v)
        gcp = [None] * nch
        ocp = [None] * nch
        for ci in range(min(3, nch)):
            gcp[ci] = pltpu.async_copy(
                table_hbm.at[idx_v.at[pl.ds(ci * GCH, GCH)]], bufs[ci % 3],
                gsem[ci % 3])
        for ci in range(nch):
            gcp[ci].wait()
            ocp[ci] = pltpu.async_copy(
                bufs[ci % 3], out_hbm.at[pl.ds(base + ci * GCH, GCH)],
                osem[ci % 3])
            nxt = ci + 3
            if nxt < nch:
                ocp[ci].wait()
                gcp[nxt] = pltpu.async_copy(
                    table_hbm.at[idx_v.at[pl.ds(nxt * GCH, GCH)]],
                    bufs[nxt % 3], gsem[nxt % 3])
        for ci in range(max(0, nch - 3), nch):
            ocp[ci].wait()

    return gather_k(hidden_states, token_pad)


def _sc_gather_pair(y_sorted, pos0, pos1):
    tok_per_w = TOKENS // NW
    nj = 2 * (tok_per_w // GCH)

    @functools.partial(
        pl.kernel,
        out_type=[jax.ShapeDtypeStruct((TOKENS, HIDDEN), jnp.float32),
                  jax.ShapeDtypeStruct((TOKENS, HIDDEN), jnp.float32)],
        mesh=plsc.VectorSubcoreMesh(**_SC_MESH),
        scratch_types=(
            [pltpu.VMEM((tok_per_w,), jnp.int32) for _ in range(2)]
            + [pltpu.VMEM((GCH, HIDDEN), jnp.float32) for _ in range(3)]
            + [pltpu.SemaphoreType.DMA for _ in range(6)]
        ),
    )
    def pair_k(y_hbm, p0_hbm, p1_hbm, out0_hbm, out1_hbm, p0_v, p1_v,
               b0, b1, b2, g0, g1, g2, o0, o1, o2):
        bufs = (b0, b1, b2)
        gsem = (g0, g1, g2)
        osem = (o0, o1, o2)
        wid = lax.axis_index("s") * NC + lax.axis_index("c")
        base = wid * tok_per_w
        pltpu.sync_copy(p0_hbm.at[pl.ds(base, tok_per_w)], p0_v)
        pltpu.sync_copy(p1_hbm.at[pl.ds(base, tok_per_w)], p1_v)
        pv = (p0_v, p1_v)

        def job(j):
            return pv[j % 2], (j // 2) * GCH

        def start_gather(j):
            idx_v, toff = job(j)
            return pltpu.async_copy(
                y_hbm.at[idx_v.at[pl.ds(toff, GCH)]], bufs[j % 3], gsem[j % 3])

        gcp = [None] * nj
        ocp = [None] * nj
        for j in range(min(3, nj)):
            gcp[j] = start_gather(j)
        for j in range(nj):
            idx_v, toff = job(j)
            out_hbm = (out0_hbm, out1_hbm)[j % 2]
            gcp[j].wait()
            ocp[j] = pltpu.async_copy(
                bufs[j % 3], out_hbm.at[pl.ds(base + toff, GCH)], osem[j % 3])
            nxt = j + 3
            if nxt < nj:
                ocp[j].wait()
                gcp[nxt] = start_gather(nxt)
        for j in range(max(0, nj - 3), nj):
            ocp[j].wait()

    return pair_k(y_sorted, pos0, pos1)


def _add_body(a_ref, b_ref, wa_ref, wb_ref, o_ref):
    o_ref[...] = (a_ref[...] * wa_ref[:, 0:1] + b_ref[...] * wb_ref[:, 0:1])


def _tc_add(a, b, wa, wb):
    blk = 256
    return pl.pallas_call(
        _add_body,
        grid=(TOKENS // blk,),
        in_specs=[pl.BlockSpec((blk, HIDDEN), lambda i: (i, 0)),
                  pl.BlockSpec((blk, HIDDEN), lambda i: (i, 0)),
                  pl.BlockSpec((blk, 128), lambda i: (i, 0)),
                  pl.BlockSpec((blk, 128), lambda i: (i, 0))],
        out_specs=pl.BlockSpec((blk, HIDDEN), lambda i: (i, 0)),
        out_shape=jax.ShapeDtypeStruct((TOKENS, HIDDEN), jnp.float32),
    )(a, b, wa, wb)


def _fused_body(meta_ref, x_ref, gw_ref, uw_ref, dw_ref, y_ref):
    @pl.when(meta_ref[2, pl.program_id(0)] == 1)
    def _():
        x = x_ref[...]
        dn = (((1,), (1,)), ((), ()))
        acc = jnp.zeros((BR, HIDDEN), jnp.float32)
        for c in range(NIT):
            gw = gw_ref[0, 0, c * IT:(c + 1) * IT, :]
            uw = uw_ref[0, 0, c * IT:(c + 1) * IT, :]
            g = lax.dot_general(x, gw, dn, preferred_element_type=jnp.float32)
            u = lax.dot_general(x, uw, dn, preferred_element_type=jnp.float32)
            h = (g * lax.logistic(g)) * u
            dwc = dw_ref[0, :, c * IT:(c + 1) * IT]
            acc = acc + lax.dot_general(h, dwc, dn,
                                        preferred_element_type=jnp.float32)
        y_ref[...] = acc


def _grouped_mlp(x_sorted, gup4, down_proj, block_meta):
    return pl.pallas_call(
        _fused_body,
        grid_spec=pltpu.PrefetchScalarGridSpec(
            num_scalar_prefetch=1,
            grid=(NB,),
            in_specs=[
                pl.BlockSpec((BR, HIDDEN), lambda b, m: (m[1, b], 0)),
                pl.BlockSpec((1, 1, INTER, HIDDEN), lambda b, m: (m[0, b], 0, 0, 0)),
                pl.BlockSpec((1, 1, INTER, HIDDEN), lambda b, m: (m[0, b], 1, 0, 0)),
                pl.BlockSpec((1, HIDDEN, INTER), lambda b, m: (m[0, b], 0, 0)),
            ],
            out_specs=pl.BlockSpec((BR, HIDDEN), lambda b, m: (m[1, b], 0)),
        ),
        out_shape=jax.ShapeDtypeStruct((NPAD, HIDDEN), jnp.float32),
    )(block_meta, x_sorted, gup4, gup4, down_proj)


def kernel(hidden_states, top_k_index, top_k_weights, gate_up_proj, down_proj):
    pos, token_pad, block_meta = _routing_metadata(top_k_index)
    w32 = top_k_weights.astype(jnp.float32)
    wa = jnp.broadcast_to(w32[:, 0:1], (TOKENS, 128))
    wb = jnp.broadcast_to(w32[:, 1:2], (TOKENS, 128))
    gup4 = gate_up_proj.reshape(NUM_EXPERTS, 2, INTER, HIDDEN)

    x_sorted = _sc_gather(hidden_states, token_pad)
    y = _grouped_mlp(x_sorted, gup4, down_proj, block_meta)
    pos2 = pos.reshape(TOKENS, TOPK)
    c0, c1 = _sc_gather_pair(y, pos2[:, 0], pos2[:, 1])
    return _tc_add(c0, c1, wa, wb)
